# Initial kernel scaffold; baseline (speedup 1.0000x reference)
import jax
import jax.numpy as jnp
from jax.experimental import pallas as pl


def kernel(x1, edge_index1, x2, edge_index2, W1_1, W1_2, W2_1, W2_2, Wg, bg):
    raise NotImplementedError("write your pallas kernel here")


# speedup vs baseline: 7.9073x; 7.9073x over previous
import functools

import jax
import jax.numpy as jnp
from jax import lax
from jax.experimental import pallas as pl
from jax.experimental.pallas import tpu as pltpu
from jax.experimental.pallas import tpu_sc as plsc

_N = 10000
_E = 320000
_D = 128
_H1 = 32
_H2 = 16

_NC = 2
_NS = 16
_NW = _NC * _NS
_L = 16
_EPW = _E // _NW
_RPS = _N // _NS
_CB = 128
_NCHUNK = _E // _CB
_CPW = _NCHUNK // _NW
_XTRA = _NCHUNK % _NW
_ZR = 640
_ZLAST = _N - (_NS - 1) * _ZR

_MESH = plsc.VectorSubcoreMesh(core_axis_name="c", subcore_axis_name="s")


@functools.partial(
    pl.kernel,
    out_type=jax.ShapeDtypeStruct((2 * _NW, _N), jnp.float32),
    mesh=_MESH,
    compiler_params=pltpu.CompilerParams(needs_layout_passes=False),
    scratch_types=[
        pltpu.VMEM((_EPW,), jnp.int32),
        pltpu.VMEM((_EPW,), jnp.int32),
        pltpu.VMEM((_N,), jnp.float32),
        pltpu.VMEM((_N,), jnp.float32),
    ],
)
def _sc_hist(src_hbm, dst_hbm, out_hbm, idx_s, idx_d, hist_s, hist_d):
    wid = lax.axis_index("s") * _NC + lax.axis_index("c")
    base = wid * _EPW
    pltpu.sync_copy(src_hbm.at[pl.ds(base, _EPW)], idx_s)
    pltpu.sync_copy(dst_hbm.at[pl.ds(base, _EPW)], idx_d)

    zeros = jnp.zeros((_L,), jnp.float32)

    def zbody(i, c):
        hist_s[pl.ds(i * _L, _L)] = zeros
        hist_d[pl.ds(i * _L, _L)] = zeros
        return c

    lax.fori_loop(0, _N // _L, zbody, 0)

    ones = jnp.ones((_L,), jnp.float32)

    def ebody(i, c):
        plsc.addupdate_scatter(hist_s, [idx_s[pl.ds(i * _L, _L)]], ones)
        plsc.addupdate_scatter(hist_d, [idx_d[pl.ds(i * _L, _L)]], ones)
        return c

    lax.fori_loop(0, _EPW // _L, ebody, 0)

    pltpu.sync_copy(hist_s, out_hbm.at[wid])
    pltpu.sync_copy(hist_d, out_hbm.at[_NW + wid])


def _make_spmm(f):
    @functools.partial(
        pl.kernel,
        out_type=jax.ShapeDtypeStruct((_NC * _N, f), jnp.float32),
        mesh=_MESH,
        compiler_params=pltpu.CompilerParams(
            needs_layout_passes=False, use_tc_tiling_on_sc=False),
        scratch_types=[
            pltpu.VMEM((_CB,), jnp.int32),
            pltpu.VMEM((_CB,), jnp.int32),
            pltpu.VMEM((_CB, f), jnp.float32),
            pltpu.VMEM((_ZR, f), jnp.float32),
            pltpu.VMEM_SHARED((_N, f), jnp.float32),
        ],
    )
    def spmm(ys_hbm, src_hbm, dst_hbm, out_hbm, sidx, didx, rows, zbuf, acc):
        cid = lax.axis_index("c")
        sid = lax.axis_index("s")
        wid = sid * _NC + cid

        zeros = jnp.zeros((_L,), jnp.float32)

        def zbody(r, c):
            for j in range(f // _L):
                zbuf[r, pl.ds(j * _L, _L)] = zeros
            return c

        lax.fori_loop(0, _ZR, zbody, 0)

        @pl.when(sid < _NS - 1)
        def _():
            pltpu.sync_copy(zbuf, acc.at[pl.ds(sid * _ZR, _ZR)])

        @pl.when(sid == _NS - 1)
        def _():
            pltpu.sync_copy(zbuf.at[pl.ds(0, _ZLAST)],
                            acc.at[pl.ds((_NS - 1) * _ZR, _ZLAST)])

        plsc.subcore_barrier()

        cnt = jnp.where(wid < _XTRA, _CPW + 1, _CPW)
        start = wid * _CPW + jnp.minimum(wid, _XTRA)

        def ebody(k, c):
            @pl.when(k < cnt)
            def _():
                chunk = start + k
                pltpu.sync_copy(src_hbm.at[chunk], sidx)
                pltpu.sync_copy(dst_hbm.at[chunk], didx)
                pltpu.sync_copy(ys_hbm.at[sidx], rows)
                pltpu.sync_copy(rows, acc.at[didx], add=True)
            return c

        lax.fori_loop(0, _CPW + 1, ebody, 0)
        plsc.subcore_barrier()

        @pl.when(sid < _NS - 1)
        def _():
            pltpu.sync_copy(acc.at[pl.ds(sid * _ZR, _ZR)],
                            out_hbm.at[pl.ds(cid * _N + sid * _ZR, _ZR)])

        @pl.when(sid == _NS - 1)
        def _():
            pltpu.sync_copy(
                acc.at[pl.ds((_NS - 1) * _ZR, _ZLAST)],
                out_hbm.at[pl.ds(cid * _N + (_NS - 1) * _ZR, _ZLAST)])

    return spmm


_spmm_h1 = _make_spmm(_H1)
_spmm_h2 = _make_spmm(_H2)


_R = 2000
_NB = _N // _R


def _tc_a_body(pt_ref, x_ref, w_ref, ys_ref, yself_ref, sc_ref):
    pt = pt_ref[...]
    deg_o = jnp.sum(pt[:, :_NW], axis=1, keepdims=True) + 1.0
    deg_i = jnp.sum(pt[:, _NW:], axis=1, keepdims=True) + 1.0
    r_o = lax.rsqrt(deg_o)
    r_i = lax.rsqrt(deg_i)
    s = r_o * r_i
    y = jnp.dot(x_ref[...], w_ref[...], preferred_element_type=jnp.float32)
    ys_ref[...] = y * r_o
    yself_ref[...] = y * s
    sc_ref[...] = jnp.concatenate([r_o, r_i, s, jnp.zeros_like(s)], axis=1)


_tc_a = pl.pallas_call(
    _tc_a_body,
    grid=(_NB,),
    in_specs=[
        pl.BlockSpec((_R, 2 * _NW), lambda i: (i, 0)),
        pl.BlockSpec((_R, _D), lambda i: (i, 0)),
        pl.BlockSpec((_D, _H1), lambda i: (0, 0)),
    ],
    out_specs=[
        pl.BlockSpec((_R, _H1), lambda i: (i, 0)),
        pl.BlockSpec((_R, _H1), lambda i: (i, 0)),
        pl.BlockSpec((_R, 4), lambda i: (i, 0)),
    ],
    out_shape=[
        jax.ShapeDtypeStruct((_N, _H1), jnp.float32),
        jax.ShapeDtypeStruct((_N, _H1), jnp.float32),
        jax.ShapeDtypeStruct((_N, 4), jnp.float32),
    ],
)


def _tc_c_body(p0_ref, p1_ref, yself_ref, sc_ref, w_ref, y2s_ref, y2self_ref):
    sc = sc_ref[...]
    h = jnp.maximum(
        sc[:, 1:2] * (p0_ref[...] + p1_ref[...]) + yself_ref[...], 0.0)
    y2 = jnp.dot(h, w_ref[...], preferred_element_type=jnp.float32)
    y2s_ref[...] = y2 * sc[:, 0:1]
    y2self_ref[...] = y2 * sc[:, 2:3]


_tc_c = pl.pallas_call(
    _tc_c_body,
    grid=(_NB,),
    in_specs=[
        pl.BlockSpec((_R, _H1), lambda i: (i, 0)),
        pl.BlockSpec((_R, _H1), lambda i: (i + _NB, 0)),
        pl.BlockSpec((_R, _H1), lambda i: (i, 0)),
        pl.BlockSpec((_R, 4), lambda i: (i, 0)),
        pl.BlockSpec((_H1, _H2), lambda i: (0, 0)),
    ],
    out_specs=[
        pl.BlockSpec((_R, _H2), lambda i: (i, 0)),
        pl.BlockSpec((_R, _H2), lambda i: (i, 0)),
    ],
    out_shape=[
        jax.ShapeDtypeStruct((_N, _H2), jnp.float32),
        jax.ShapeDtypeStruct((_N, _H2), jnp.float32),
    ],
)


def _tc_d_body(p0_ref, p1_ref, y2self_ref, sc_ref, wg_ref, bg_ref,
               z_ref, head_ref):
    sc = sc_ref[...]
    z = sc[:, 1:2] * (p0_ref[...] + p1_ref[...]) + y2self_ref[...]
    z_ref[...] = z
    head_ref[...] = (
        jnp.dot(z, wg_ref[...], preferred_element_type=jnp.float32)
        + bg_ref[...])


_tc_d = pl.pallas_call(
    _tc_d_body,
    grid=(_NB,),
    in_specs=[
        pl.BlockSpec((_R, _H2), lambda i: (i, 0)),
        pl.BlockSpec((_R, _H2), lambda i: (i + _NB, 0)),
        pl.BlockSpec((_R, _H2), lambda i: (i, 0)),
        pl.BlockSpec((_R, 4), lambda i: (i, 0)),
        pl.BlockSpec((_H2, _H2), lambda i: (0, 0)),
        pl.BlockSpec((1, _H2), lambda i: (0, 0)),
    ],
    out_specs=[
        pl.BlockSpec((_R, _H2), lambda i: (i, 0)),
        pl.BlockSpec((_R, _H2), lambda i: (i, 0)),
    ],
    out_shape=[
        jax.ShapeDtypeStruct((_N, _H2), jnp.float32),
        jax.ShapeDtypeStruct((_N, _H2), jnp.float32),
    ],
)


_BM = 200


def _tc_rec_body(a_ref, b_ref, o_ref):
    o_ref[...] = lax.dot_general(
        a_ref[...], b_ref[...], (((1,), (1,)), ((), ())),
        preferred_element_type=jnp.float32)


_tc_rec = pl.pallas_call(
    _tc_rec_body,
    grid=(_N // _BM,),
    in_specs=[
        pl.BlockSpec((_BM, _H2), lambda i: (i, 0)),
        pl.BlockSpec((_N, _H2), lambda i: (0, 0)),
    ],
    out_specs=pl.BlockSpec((_BM, _N), lambda i: (i, 0)),
    out_shape=jax.ShapeDtypeStruct((_N, _N), jnp.float32),
)


def _encode(x, ei, wa, wb, wg, bg2):
    src = ei[0]
    dst = ei[1]
    hist = _sc_hist(src, dst)
    pt = hist.T
    ys, yself, scales = _tc_a(pt, x, wa)
    src_r = src.reshape(_NCHUNK, _CB)
    dst_r = dst.reshape(_NCHUNK, _CB)
    p1 = _spmm_h1(ys, src_r, dst_r)
    y2s, y2self = _tc_c(p1, p1, yself, scales, wb)
    p2 = _spmm_h2(y2s, src_r, dst_r)
    z, head = _tc_d(p2, p2, y2self, scales, wg, bg2)
    return z, head


def kernel(x1, edge_index1, x2, edge_index2, W1_1, W1_2, W2_1, W2_2, Wg, bg):
    bg2 = bg.reshape(1, _H2)
    z1, head1 = _encode(x1, edge_index1, W1_1, W1_2, Wg, bg2)
    z2, _ = _encode(x2, edge_index2, W2_1, W2_2, Wg, bg2)
    rec1 = _tc_rec(z1, z1).reshape(-1)
    rec2 = _tc_rec(z2, z2).reshape(-1)
    return rec1, rec2, head1


# speedup vs baseline: 10.4157x; 1.3172x over previous
import functools

import jax
import jax.numpy as jnp
from jax import lax
from jax.experimental import pallas as pl
from jax.experimental.pallas import tpu as pltpu
from jax.experimental.pallas import tpu_sc as plsc

_N = 10000
_E = 320000
_D = 128
_H1 = 32
_H2 = 16

_NC = 2
_NS = 16
_NW = _NC * _NS
_L = 16
_EPW = _E // _NW
_RPS = _N // _NS
_CB = 128
_NCHUNK = _E // _CB
_CPW = _NCHUNK // _NW
_XTRA = _NCHUNK % _NW
_ZR = 640
_ZLAST = _N - (_NS - 1) * _ZR

_MESH = plsc.VectorSubcoreMesh(core_axis_name="c", subcore_axis_name="s")


@functools.partial(
    pl.kernel,
    out_type=jax.ShapeDtypeStruct((2 * _NW, _N), jnp.float32),
    mesh=_MESH,
    compiler_params=pltpu.CompilerParams(needs_layout_passes=False),
    scratch_types=[
        pltpu.VMEM((_EPW,), jnp.int32),
        pltpu.VMEM((_EPW,), jnp.int32),
        pltpu.VMEM((_N,), jnp.float32),
        pltpu.VMEM((_N,), jnp.float32),
    ],
)
def _sc_hist(src_hbm, dst_hbm, out_hbm, idx_s, idx_d, hist_s, hist_d):
    wid = lax.axis_index("s") * _NC + lax.axis_index("c")
    base = wid * _EPW
    pltpu.sync_copy(src_hbm.at[pl.ds(base, _EPW)], idx_s)
    pltpu.sync_copy(dst_hbm.at[pl.ds(base, _EPW)], idx_d)

    zeros = jnp.zeros((_L,), jnp.float32)

    def zbody(i, c):
        hist_s[pl.ds(i * _L, _L)] = zeros
        hist_d[pl.ds(i * _L, _L)] = zeros
        return c

    lax.fori_loop(0, _N // _L, zbody, 0)

    ones = jnp.ones((_L,), jnp.float32)

    def ebody(i, c):
        plsc.addupdate_scatter(hist_s, [idx_s[pl.ds(i * _L, _L)]], ones)
        plsc.addupdate_scatter(hist_d, [idx_d[pl.ds(i * _L, _L)]], ones)
        return c

    lax.fori_loop(0, _EPW // _L, ebody, 0)

    pltpu.sync_copy(hist_s, out_hbm.at[wid])
    pltpu.sync_copy(hist_d, out_hbm.at[_NW + wid])


def _make_spmm(f):
    @functools.partial(
        pl.kernel,
        out_type=jax.ShapeDtypeStruct((_NC * _N, f), jnp.float32),
        mesh=_MESH,
        compiler_params=pltpu.CompilerParams(
            needs_layout_passes=False, use_tc_tiling_on_sc=False),
        scratch_types=[
            pltpu.VMEM((_CPW + 1, _CB), jnp.int32),
            pltpu.VMEM((_CPW + 1, _CB), jnp.int32),
            pltpu.VMEM((_CB, f), jnp.float32),
            pltpu.VMEM((_CB, f), jnp.float32),
            pltpu.VMEM((_ZR, f), jnp.float32),
            pltpu.VMEM_SHARED((_N, f), jnp.float32),
            pltpu.SemaphoreType.DMA,
            pltpu.SemaphoreType.DMA,
        ],
    )
    def spmm(ys_hbm, src_hbm, dst_hbm, out_hbm, sidx, didx,
             rows_a, rows_b, zbuf, acc, sem_a, sem_b):
        cid = lax.axis_index("c")
        sid = lax.axis_index("s")
        wid = sid * _NC + cid

        zeros = jnp.zeros((_L,), jnp.float32)

        def zbody(r, c):
            for j in range(f // _L):
                zbuf[r, pl.ds(j * _L, _L)] = zeros
            return c

        lax.fori_loop(0, _ZR, zbody, 0)

        @pl.when(sid < _NS - 1)
        def _():
            pltpu.sync_copy(zbuf, acc.at[pl.ds(sid * _ZR, _ZR)])

        @pl.when(sid == _NS - 1)
        def _():
            pltpu.sync_copy(zbuf.at[pl.ds(0, _ZLAST)],
                            acc.at[pl.ds((_NS - 1) * _ZR, _ZLAST)])

        cnt = jnp.where(wid < _XTRA, _CPW + 1, _CPW)
        start = wid * _CPW + jnp.minimum(wid, _XTRA)
        pltpu.sync_copy(src_hbm.at[pl.ds(start, _CPW)],
                        sidx.at[pl.ds(0, _CPW)])
        pltpu.sync_copy(dst_hbm.at[pl.ds(start, _CPW)],
                        didx.at[pl.ds(0, _CPW)])

        @pl.when(wid < _XTRA)
        def _():
            pltpu.sync_copy(src_hbm.at[pl.ds(start + _CPW, 1)],
                            sidx.at[pl.ds(_CPW, 1)])
            pltpu.sync_copy(dst_hbm.at[pl.ds(start + _CPW, 1)],
                            didx.at[pl.ds(_CPW, 1)])

        plsc.subcore_barrier()

        bufs = (rows_a, rows_b)
        sems = (sem_a, sem_b)

        def issue(k, b):
            @pl.when(k < cnt)
            def _():
                pltpu.async_copy(ys_hbm.at[sidx.at[k]], bufs[b], sems[b])

        def drain(k, b):
            @pl.when(k < cnt)
            def _():
                pltpu.make_async_copy(
                    ys_hbm.at[sidx.at[k]], bufs[b], sems[b]).wait()
                pltpu.sync_copy(bufs[b], acc.at[didx.at[k]], add=True)

        issue(0, 0)

        def ebody(i, c):
            k0 = 2 * i
            issue(k0 + 1, 1)
            drain(k0, 0)
            issue(k0 + 2, 0)
            drain(k0 + 1, 1)
            return c

        lax.fori_loop(0, (_CPW + 2) // 2, ebody, 0)
        plsc.subcore_barrier()

        @pl.when(sid < _NS - 1)
        def _():
            pltpu.sync_copy(acc.at[pl.ds(sid * _ZR, _ZR)],
                            out_hbm.at[pl.ds(cid * _N + sid * _ZR, _ZR)])

        @pl.when(sid == _NS - 1)
        def _():
            pltpu.sync_copy(
                acc.at[pl.ds((_NS - 1) * _ZR, _ZLAST)],
                out_hbm.at[pl.ds(cid * _N + (_NS - 1) * _ZR, _ZLAST)])

    return spmm


_spmm_h1 = _make_spmm(_H1)
_spmm_h2 = _make_spmm(_H2)


_R = 2000
_NB = _N // _R


def _tc_a_body(pt_ref, x_ref, w_ref, ys_ref, yself_ref, sc_ref):
    pt = pt_ref[...]
    deg_o = jnp.sum(pt[:, :_NW], axis=1, keepdims=True) + 1.0
    deg_i = jnp.sum(pt[:, _NW:], axis=1, keepdims=True) + 1.0
    r_o = lax.rsqrt(deg_o)
    r_i = lax.rsqrt(deg_i)
    s = r_o * r_i
    y = jnp.dot(x_ref[...], w_ref[...], preferred_element_type=jnp.float32)
    ys_ref[...] = y * r_o
    yself_ref[...] = y * s
    sc_ref[...] = jnp.concatenate([r_o, r_i, s, jnp.zeros_like(s)], axis=1)


_tc_a = pl.pallas_call(
    _tc_a_body,
    grid=(_NB,),
    in_specs=[
        pl.BlockSpec((_R, 2 * _NW), lambda i: (i, 0)),
        pl.BlockSpec((_R, _D), lambda i: (i, 0)),
        pl.BlockSpec((_D, _H1), lambda i: (0, 0)),
    ],
    out_specs=[
        pl.BlockSpec((_R, _H1), lambda i: (i, 0)),
        pl.BlockSpec((_R, _H1), lambda i: (i, 0)),
        pl.BlockSpec((_R, 4), lambda i: (i, 0)),
    ],
    out_shape=[
        jax.ShapeDtypeStruct((_N, _H1), jnp.float32),
        jax.ShapeDtypeStruct((_N, _H1), jnp.float32),
        jax.ShapeDtypeStruct((_N, 4), jnp.float32),
    ],
)


def _tc_c_body(p0_ref, p1_ref, yself_ref, sc_ref, w_ref, y2s_ref, y2self_ref):
    sc = sc_ref[...]
    h = jnp.maximum(
        sc[:, 1:2] * (p0_ref[...] + p1_ref[...]) + yself_ref[...], 0.0)
    y2 = jnp.dot(h, w_ref[...], preferred_element_type=jnp.float32)
    y2s_ref[...] = y2 * sc[:, 0:1]
    y2self_ref[...] = y2 * sc[:, 2:3]


_tc_c = pl.pallas_call(
    _tc_c_body,
    grid=(_NB,),
    in_specs=[
        pl.BlockSpec((_R, _H1), lambda i: (i, 0)),
        pl.BlockSpec((_R, _H1), lambda i: (i + _NB, 0)),
        pl.BlockSpec((_R, _H1), lambda i: (i, 0)),
        pl.BlockSpec((_R, 4), lambda i: (i, 0)),
        pl.BlockSpec((_H1, _H2), lambda i: (0, 0)),
    ],
    out_specs=[
        pl.BlockSpec((_R, _H2), lambda i: (i, 0)),
        pl.BlockSpec((_R, _H2), lambda i: (i, 0)),
    ],
    out_shape=[
        jax.ShapeDtypeStruct((_N, _H2), jnp.float32),
        jax.ShapeDtypeStruct((_N, _H2), jnp.float32),
    ],
)


def _tc_d_body(p0_ref, p1_ref, y2self_ref, sc_ref, wg_ref, bg_ref,
               z_ref, head_ref):
    sc = sc_ref[...]
    z = sc[:, 1:2] * (p0_ref[...] + p1_ref[...]) + y2self_ref[...]
    z_ref[...] = z
    head_ref[...] = (
        jnp.dot(z, wg_ref[...], preferred_element_type=jnp.float32)
        + bg_ref[...])


_tc_d = pl.pallas_call(
    _tc_d_body,
    grid=(_NB,),
    in_specs=[
        pl.BlockSpec((_R, _H2), lambda i: (i, 0)),
        pl.BlockSpec((_R, _H2), lambda i: (i + _NB, 0)),
        pl.BlockSpec((_R, _H2), lambda i: (i, 0)),
        pl.BlockSpec((_R, 4), lambda i: (i, 0)),
        pl.BlockSpec((_H2, _H2), lambda i: (0, 0)),
        pl.BlockSpec((1, _H2), lambda i: (0, 0)),
    ],
    out_specs=[
        pl.BlockSpec((_R, _H2), lambda i: (i, 0)),
        pl.BlockSpec((_R, _H2), lambda i: (i, 0)),
    ],
    out_shape=[
        jax.ShapeDtypeStruct((_N, _H2), jnp.float32),
        jax.ShapeDtypeStruct((_N, _H2), jnp.float32),
    ],
)


_BM = 200


def _tc_rec_body(a_ref, b_ref, o_ref):
    o_ref[...] = lax.dot_general(
        a_ref[...], b_ref[...], (((1,), (1,)), ((), ())),
        preferred_element_type=jnp.float32)


_tc_rec = pl.pallas_call(
    _tc_rec_body,
    grid=(_N // _BM,),
    in_specs=[
        pl.BlockSpec((_BM, _H2), lambda i: (i, 0)),
        pl.BlockSpec((_N, _H2), lambda i: (0, 0)),
    ],
    out_specs=pl.BlockSpec((_BM, _N), lambda i: (i, 0)),
    out_shape=jax.ShapeDtypeStruct((_N, _N), jnp.float32),
)


def _encode(x, ei, wa, wb, wg, bg2):
    src = ei[0]
    dst = ei[1]
    hist = _sc_hist(src, dst)
    pt = hist.T
    ys, yself, scales = _tc_a(pt, x, wa)
    src_r = src.reshape(_NCHUNK, _CB)
    dst_r = dst.reshape(_NCHUNK, _CB)
    p1 = _spmm_h1(ys, src_r, dst_r)
    y2s, y2self = _tc_c(p1, p1, yself, scales, wb)
    p2 = _spmm_h2(y2s, src_r, dst_r)
    z, head = _tc_d(p2, p2, y2self, scales, wg, bg2)
    return z, head


def kernel(x1, edge_index1, x2, edge_index2, W1_1, W1_2, W2_1, W2_2, Wg, bg):
    bg2 = bg.reshape(1, _H2)
    z1, head1 = _encode(x1, edge_index1, W1_1, W1_2, Wg, bg2)
    z2, _ = _encode(x2, edge_index2, W2_1, W2_2, Wg, bg2)
    rec1 = _tc_rec(z1, z1).reshape(-1)
    rec2 = _tc_rec(z2, z2).reshape(-1)
    return rec1, rec2, head1


# speedup vs baseline: 10.4255x; 1.0009x over previous
import functools

import jax
import jax.numpy as jnp
from jax import lax
from jax.experimental import pallas as pl
from jax.experimental.pallas import tpu as pltpu
from jax.experimental.pallas import tpu_sc as plsc

_N = 10000
_E = 320000
_D = 128
_H1 = 32
_H2 = 16

_NC = 2
_NS = 16
_NW = _NC * _NS
_L = 16
_EPW = _E // _NW
_RPS = _N // _NS
_CB = 128
_NCHUNK = _E // _CB
_CPW = _NCHUNK // _NW
_XTRA = _NCHUNK % _NW
_ZR = 640
_ZLAST = _N - (_NS - 1) * _ZR

_MESH = plsc.VectorSubcoreMesh(core_axis_name="c", subcore_axis_name="s")


@functools.partial(
    pl.kernel,
    out_type=jax.ShapeDtypeStruct((2 * _NW, _N), jnp.float32),
    mesh=_MESH,
    compiler_params=pltpu.CompilerParams(needs_layout_passes=False),
    scratch_types=[
        pltpu.VMEM((_EPW,), jnp.int32),
        pltpu.VMEM((_EPW,), jnp.int32),
        pltpu.VMEM((_N,), jnp.float32),
        pltpu.VMEM((_N,), jnp.float32),
    ],
)
def _sc_hist(src_hbm, dst_hbm, out_hbm, idx_s, idx_d, hist_s, hist_d):
    wid = lax.axis_index("s") * _NC + lax.axis_index("c")
    base = wid * _EPW
    pltpu.sync_copy(src_hbm.at[pl.ds(base, _EPW)], idx_s)
    pltpu.sync_copy(dst_hbm.at[pl.ds(base, _EPW)], idx_d)

    zeros = jnp.zeros((_L,), jnp.float32)

    def zbody(i, c):
        hist_s[pl.ds(i * _L, _L)] = zeros
        hist_d[pl.ds(i * _L, _L)] = zeros
        return c

    lax.fori_loop(0, _N // _L, zbody, 0)

    ones = jnp.ones((_L,), jnp.float32)

    def ebody(i, c):
        plsc.addupdate_scatter(hist_s, [idx_s[pl.ds(i * _L, _L)]], ones)
        plsc.addupdate_scatter(hist_d, [idx_d[pl.ds(i * _L, _L)]], ones)
        return c

    lax.fori_loop(0, _EPW // _L, ebody, 0)

    pltpu.sync_copy(hist_s, out_hbm.at[wid])
    pltpu.sync_copy(hist_d, out_hbm.at[_NW + wid])


def _make_spmm(f):
    @functools.partial(
        pl.kernel,
        out_type=jax.ShapeDtypeStruct((_NC * _N, f), jnp.float32),
        mesh=_MESH,
        compiler_params=pltpu.CompilerParams(
            needs_layout_passes=False, use_tc_tiling_on_sc=False),
        scratch_types=[
            pltpu.VMEM((_CPW + 1, _CB), jnp.int32),
            pltpu.VMEM((_CPW + 1, _CB), jnp.int32),
            pltpu.VMEM((_CB, f), jnp.float32),
            pltpu.VMEM((_CB, f), jnp.float32),
            pltpu.VMEM((_ZR, f), jnp.float32),
            pltpu.VMEM_SHARED((_N, f), jnp.float32),
            pltpu.SemaphoreType.DMA,
            pltpu.SemaphoreType.DMA,
        ],
    )
    def spmm(ys_hbm, src_hbm, dst_hbm, out_hbm, sidx, didx,
             rows_a, rows_b, zbuf, acc, sem_a, sem_b):
        cid = lax.axis_index("c")
        sid = lax.axis_index("s")
        wid = sid * _NC + cid

        zeros = jnp.zeros((_L,), jnp.float32)

        def zbody(r, c):
            for j in range(f // _L):
                zbuf[r, pl.ds(j * _L, _L)] = zeros
            return c

        lax.fori_loop(0, _ZR, zbody, 0)

        @pl.when(sid < _NS - 1)
        def _():
            pltpu.sync_copy(zbuf, acc.at[pl.ds(sid * _ZR, _ZR)])

        @pl.when(sid == _NS - 1)
        def _():
            pltpu.sync_copy(zbuf.at[pl.ds(0, _ZLAST)],
                            acc.at[pl.ds((_NS - 1) * _ZR, _ZLAST)])

        cnt = jnp.where(wid < _XTRA, _CPW + 1, _CPW)
        start = wid * _CPW + jnp.minimum(wid, _XTRA)
        pltpu.sync_copy(src_hbm.at[pl.ds(start, _CPW)],
                        sidx.at[pl.ds(0, _CPW)])
        pltpu.sync_copy(dst_hbm.at[pl.ds(start, _CPW)],
                        didx.at[pl.ds(0, _CPW)])

        @pl.when(wid < _XTRA)
        def _():
            pltpu.sync_copy(src_hbm.at[pl.ds(start + _CPW, 1)],
                            sidx.at[pl.ds(_CPW, 1)])
            pltpu.sync_copy(dst_hbm.at[pl.ds(start + _CPW, 1)],
                            didx.at[pl.ds(_CPW, 1)])

        plsc.subcore_barrier()

        bufs = (rows_a, rows_b)
        sems = (sem_a, sem_b)

        def issue(k, b):
            @pl.when(k < cnt)
            def _():
                pltpu.async_copy(ys_hbm.at[sidx.at[k]], bufs[b], sems[b])

        def drain(k, b):
            @pl.when(k < cnt)
            def _():
                pltpu.make_async_copy(
                    ys_hbm.at[sidx.at[k]], bufs[b], sems[b]).wait()
                pltpu.sync_copy(bufs[b], acc.at[didx.at[k]], add=True)

        issue(0, 0)

        def ebody(i, c):
            k0 = 2 * i
            issue(k0 + 1, 1)
            drain(k0, 0)
            issue(k0 + 2, 0)
            drain(k0 + 1, 1)
            return c

        lax.fori_loop(0, (_CPW + 2) // 2, ebody, 0)
        plsc.subcore_barrier()

        @pl.when(sid < _NS - 1)
        def _():
            pltpu.sync_copy(acc.at[pl.ds(sid * _ZR, _ZR)],
                            out_hbm.at[pl.ds(cid * _N + sid * _ZR, _ZR)])

        @pl.when(sid == _NS - 1)
        def _():
            pltpu.sync_copy(
                acc.at[pl.ds((_NS - 1) * _ZR, _ZLAST)],
                out_hbm.at[pl.ds(cid * _N + (_NS - 1) * _ZR, _ZLAST)])

    return spmm


_spmm_h1 = _make_spmm(_H1)
_spmm_h2 = _make_spmm(_H2)


_R = 2000
_NB = _N // _R


def _tc_a_body(pt_ref, x_ref, w_ref, ys_ref, yself_ref, sc_ref):
    pt = pt_ref[...]
    deg_o = jnp.sum(pt[:, :_NW], axis=1, keepdims=True) + 1.0
    deg_i = jnp.sum(pt[:, _NW:], axis=1, keepdims=True) + 1.0
    r_o = lax.rsqrt(deg_o)
    r_i = lax.rsqrt(deg_i)
    s = r_o * r_i
    y = jnp.dot(x_ref[...], w_ref[...], preferred_element_type=jnp.float32)
    ys_ref[...] = y * r_o
    yself_ref[...] = y * s
    sc_ref[...] = jnp.concatenate([r_o, r_i, s, jnp.zeros_like(s)], axis=1)


_tc_a = pl.pallas_call(
    _tc_a_body,
    grid=(_NB,),
    in_specs=[
        pl.BlockSpec((_R, 2 * _NW), lambda i: (i, 0)),
        pl.BlockSpec((_R, _D), lambda i: (i, 0)),
        pl.BlockSpec((_D, _H1), lambda i: (0, 0)),
    ],
    out_specs=[
        pl.BlockSpec((_R, _H1), lambda i: (i, 0)),
        pl.BlockSpec((_R, _H1), lambda i: (i, 0)),
        pl.BlockSpec((_R, 4), lambda i: (i, 0)),
    ],
    out_shape=[
        jax.ShapeDtypeStruct((_N, _H1), jnp.float32),
        jax.ShapeDtypeStruct((_N, _H1), jnp.float32),
        jax.ShapeDtypeStruct((_N, 4), jnp.float32),
    ],
)


def _tc_c_body(p0_ref, p1_ref, yself_ref, sc_ref, w_ref, y2s_ref, y2self_ref):
    sc = sc_ref[...]
    h = jnp.maximum(
        sc[:, 1:2] * (p0_ref[...] + p1_ref[...]) + yself_ref[...], 0.0)
    y2 = jnp.dot(h, w_ref[...], preferred_element_type=jnp.float32)
    y2s_ref[...] = y2 * sc[:, 0:1]
    y2self_ref[...] = y2 * sc[:, 2:3]


_tc_c = pl.pallas_call(
    _tc_c_body,
    grid=(_NB,),
    in_specs=[
        pl.BlockSpec((_R, _H1), lambda i: (i, 0)),
        pl.BlockSpec((_R, _H1), lambda i: (i + _NB, 0)),
        pl.BlockSpec((_R, _H1), lambda i: (i, 0)),
        pl.BlockSpec((_R, 4), lambda i: (i, 0)),
        pl.BlockSpec((_H1, _H2), lambda i: (0, 0)),
    ],
    out_specs=[
        pl.BlockSpec((_R, _H2), lambda i: (i, 0)),
        pl.BlockSpec((_R, _H2), lambda i: (i, 0)),
    ],
    out_shape=[
        jax.ShapeDtypeStruct((_N, _H2), jnp.float32),
        jax.ShapeDtypeStruct((_N, _H2), jnp.float32),
    ],
)


def _tc_d_body(p0_ref, p1_ref, y2self_ref, sc_ref, wg_ref, bg_ref,
               z_ref, head_ref):
    sc = sc_ref[...]
    z = sc[:, 1:2] * (p0_ref[...] + p1_ref[...]) + y2self_ref[...]
    z_ref[...] = z
    head_ref[...] = (
        jnp.dot(z, wg_ref[...], preferred_element_type=jnp.float32)
        + bg_ref[...])


_tc_d = pl.pallas_call(
    _tc_d_body,
    grid=(_NB,),
    in_specs=[
        pl.BlockSpec((_R, _H2), lambda i: (i, 0)),
        pl.BlockSpec((_R, _H2), lambda i: (i + _NB, 0)),
        pl.BlockSpec((_R, _H2), lambda i: (i, 0)),
        pl.BlockSpec((_R, 4), lambda i: (i, 0)),
        pl.BlockSpec((_H2, _H2), lambda i: (0, 0)),
        pl.BlockSpec((1, _H2), lambda i: (0, 0)),
    ],
    out_specs=[
        pl.BlockSpec((_R, _H2), lambda i: (i, 0)),
        pl.BlockSpec((_R, _H2), lambda i: (i, 0)),
    ],
    out_shape=[
        jax.ShapeDtypeStruct((_N, _H2), jnp.float32),
        jax.ShapeDtypeStruct((_N, _H2), jnp.float32),
    ],
)


_BM = 400


def _tc_rec_body(a_ref, b_ref, o_ref):
    o_ref[...] = lax.dot_general(
        a_ref[...], b_ref[...], (((1,), (1,)), ((), ())),
        preferred_element_type=jnp.float32)


_tc_rec = pl.pallas_call(
    _tc_rec_body,
    grid=(_N // _BM,),
    in_specs=[
        pl.BlockSpec((_BM, _H2), lambda i: (i, 0)),
        pl.BlockSpec((_N, _H2), lambda i: (0, 0)),
    ],
    out_specs=pl.BlockSpec((_BM, _N), lambda i: (i, 0)),
    out_shape=jax.ShapeDtypeStruct((_N, _N), jnp.float32),
)


def _encode(x, ei, wa, wb, wg, bg2):
    src = ei[0]
    dst = ei[1]
    hist = _sc_hist(src, dst)
    pt = hist.T
    ys, yself, scales = _tc_a(pt, x, wa)
    src_r = src.reshape(_NCHUNK, _CB)
    dst_r = dst.reshape(_NCHUNK, _CB)
    p1 = _spmm_h1(ys, src_r, dst_r)
    y2s, y2self = _tc_c(p1, p1, yself, scales, wb)
    p2 = _spmm_h2(y2s, src_r, dst_r)
    z, head = _tc_d(p2, p2, y2self, scales, wg, bg2)
    return z, head


def kernel(x1, edge_index1, x2, edge_index2, W1_1, W1_2, W2_1, W2_2, Wg, bg):
    bg2 = bg.reshape(1, _H2)
    z1, head1 = _encode(x1, edge_index1, W1_1, W1_2, Wg, bg2)
    z2, _ = _encode(x2, edge_index2, W2_1, W2_2, Wg, bg2)
    rec1 = _tc_rec(z1, z1).reshape(-1)
    rec2 = _tc_rec(z2, z2).reshape(-1)
    return rec1, rec2, head1


# speedup vs baseline: 10.9318x; 1.0486x over previous
import functools

import jax
import jax.numpy as jnp
from jax import lax
from jax.experimental import pallas as pl
from jax.experimental.pallas import tpu as pltpu
from jax.experimental.pallas import tpu_sc as plsc

_N = 10000
_E = 320000
_D = 128
_H1 = 32
_H2 = 16

_NC = 2
_NS = 16
_NW = _NC * _NS
_L = 16
_EPW = _E // _NW
_RPS = _N // _NS
_CB = 128
_NCHUNK = _E // _CB
_CPW = _NCHUNK // _NW
_XTRA = _NCHUNK % _NW
_ZR = 640
_ZLAST = _N - (_NS - 1) * _ZR

_MESH = plsc.VectorSubcoreMesh(core_axis_name="c", subcore_axis_name="s")


@functools.partial(
    pl.kernel,
    out_type=jax.ShapeDtypeStruct((2 * _NW, _N), jnp.float32),
    mesh=_MESH,
    compiler_params=pltpu.CompilerParams(needs_layout_passes=False),
    scratch_types=[
        pltpu.VMEM((_EPW,), jnp.int32),
        pltpu.VMEM((_EPW,), jnp.int32),
        pltpu.VMEM((_N,), jnp.float32),
        pltpu.VMEM((_N,), jnp.float32),
    ],
)
def _sc_hist(src_hbm, dst_hbm, out_hbm, idx_s, idx_d, hist_s, hist_d):
    wid = lax.axis_index("s") * _NC + lax.axis_index("c")
    base = wid * _EPW
    pltpu.sync_copy(src_hbm.at[pl.ds(base, _EPW)], idx_s)
    pltpu.sync_copy(dst_hbm.at[pl.ds(base, _EPW)], idx_d)

    zeros = jnp.zeros((_L,), jnp.float32)

    def zbody(i, c):
        hist_s[pl.ds(i * _L, _L)] = zeros
        hist_d[pl.ds(i * _L, _L)] = zeros
        return c

    lax.fori_loop(0, _N // _L, zbody, 0)

    ones = jnp.ones((_L,), jnp.float32)

    def ebody(i, c):
        plsc.addupdate_scatter(hist_s, [idx_s[pl.ds(i * _L, _L)]], ones)
        plsc.addupdate_scatter(hist_d, [idx_d[pl.ds(i * _L, _L)]], ones)
        return c

    lax.fori_loop(0, _EPW // _L, ebody, 0)

    pltpu.sync_copy(hist_s, out_hbm.at[wid])
    pltpu.sync_copy(hist_d, out_hbm.at[_NW + wid])


def _make_spmm(f):
    @functools.partial(
        pl.kernel,
        out_type=jax.ShapeDtypeStruct((_NC * _N, f), jnp.float32),
        mesh=_MESH,
        compiler_params=pltpu.CompilerParams(
            needs_layout_passes=False, use_tc_tiling_on_sc=False),
        scratch_types=[
            pltpu.VMEM((_CPW + 1, _CB), jnp.int32),
            pltpu.VMEM((_CPW + 1, _CB), jnp.int32),
            pltpu.VMEM((4, _CB, f), jnp.float32),
            pltpu.VMEM((_ZR, f), jnp.float32),
            pltpu.VMEM_SHARED((_N, f), jnp.float32),
            [pltpu.SemaphoreType.DMA] * 4,
            [pltpu.SemaphoreType.DMA] * 4,
        ],
    )
    def spmm(ys_hbm, src_hbm, dst_hbm, out_hbm, sidx, didx,
             rows, zbuf, acc, gsem, ssem):
        cid = lax.axis_index("c")
        sid = lax.axis_index("s")
        wid = sid * _NC + cid

        zeros = jnp.zeros((_L,), jnp.float32)

        def zbody(r, c):
            for j in range(f // _L):
                zbuf[r, pl.ds(j * _L, _L)] = zeros
            return c

        lax.fori_loop(0, _ZR, zbody, 0)

        @pl.when(sid < _NS - 1)
        def _():
            pltpu.sync_copy(zbuf, acc.at[pl.ds(sid * _ZR, _ZR)])

        @pl.when(sid == _NS - 1)
        def _():
            pltpu.sync_copy(zbuf.at[pl.ds(0, _ZLAST)],
                            acc.at[pl.ds((_NS - 1) * _ZR, _ZLAST)])

        cnt = jnp.where(wid < _XTRA, _CPW + 1, _CPW)
        start = wid * _CPW + jnp.minimum(wid, _XTRA)
        pltpu.sync_copy(src_hbm.at[pl.ds(start, _CPW)],
                        sidx.at[pl.ds(0, _CPW)])
        pltpu.sync_copy(dst_hbm.at[pl.ds(start, _CPW)],
                        didx.at[pl.ds(0, _CPW)])

        @pl.when(wid < _XTRA)
        def _():
            pltpu.sync_copy(src_hbm.at[pl.ds(start + _CPW, 1)],
                            sidx.at[pl.ds(_CPW, 1)])
            pltpu.sync_copy(dst_hbm.at[pl.ds(start + _CPW, 1)],
                            didx.at[pl.ds(_CPW, 1)])

        plsc.subcore_barrier()

        def gather(k, b):
            return pltpu.make_async_copy(
                ys_hbm.at[sidx.at[k]], rows.at[b], gsem[b])

        def scatter(k, b):
            return pltpu.make_async_copy(
                rows.at[b], acc.at[didx.at[k]], ssem[b])

        def issue(k, b):
            @pl.when(k < cnt)
            def _():
                @pl.when(k >= 4)
                def _():
                    scatter(k - 4, b).wait()
                gather(k, b).start()

        def drain(k, b):
            @pl.when(k < cnt)
            def _():
                gather(k, b).wait()
                pltpu.async_copy(rows.at[b], acc.at[didx.at[k]],
                                 ssem[b], add=True)

        for b in range(4):
            issue(b, b)

        def ebody(i, c):
            k0 = 4 * i
            for b in range(4):
                drain(k0 + b, b)
                issue(k0 + b + 4, b)
            return c

        lax.fori_loop(0, (_CPW + 4) // 4, ebody, 0)

        for b in range(4):
            scatter(0, b).wait()

        plsc.subcore_barrier()

        @pl.when(sid < _NS - 1)
        def _():
            pltpu.sync_copy(acc.at[pl.ds(sid * _ZR, _ZR)],
                            out_hbm.at[pl.ds(cid * _N + sid * _ZR, _ZR)])

        @pl.when(sid == _NS - 1)
        def _():
            pltpu.sync_copy(
                acc.at[pl.ds((_NS - 1) * _ZR, _ZLAST)],
                out_hbm.at[pl.ds(cid * _N + (_NS - 1) * _ZR, _ZLAST)])

    return spmm


_spmm_h1 = _make_spmm(_H1)
_spmm_h2 = _make_spmm(_H2)


_R = 2000
_NB = _N // _R


def _tc_a_body(pt_ref, x_ref, w_ref, ys_ref, yself_ref, sc_ref):
    pt = pt_ref[...]
    deg_o = jnp.sum(pt[:, :_NW], axis=1, keepdims=True) + 1.0
    deg_i = jnp.sum(pt[:, _NW:], axis=1, keepdims=True) + 1.0
    r_o = lax.rsqrt(deg_o)
    r_i = lax.rsqrt(deg_i)
    s = r_o * r_i
    y = jnp.dot(x_ref[...], w_ref[...], preferred_element_type=jnp.float32)
    ys_ref[...] = y * r_o
    yself_ref[...] = y * s
    sc_ref[...] = jnp.concatenate([r_o, r_i, s, jnp.zeros_like(s)], axis=1)


_tc_a = pl.pallas_call(
    _tc_a_body,
    grid=(_NB,),
    in_specs=[
        pl.BlockSpec((_R, 2 * _NW), lambda i: (i, 0)),
        pl.BlockSpec((_R, _D), lambda i: (i, 0)),
        pl.BlockSpec((_D, _H1), lambda i: (0, 0)),
    ],
    out_specs=[
        pl.BlockSpec((_R, _H1), lambda i: (i, 0)),
        pl.BlockSpec((_R, _H1), lambda i: (i, 0)),
        pl.BlockSpec((_R, 4), lambda i: (i, 0)),
    ],
    out_shape=[
        jax.ShapeDtypeStruct((_N, _H1), jnp.float32),
        jax.ShapeDtypeStruct((_N, _H1), jnp.float32),
        jax.ShapeDtypeStruct((_N, 4), jnp.float32),
    ],
)


def _tc_c_body(p0_ref, p1_ref, yself_ref, sc_ref, w_ref, y2s_ref, y2self_ref):
    sc = sc_ref[...]
    h = jnp.maximum(
        sc[:, 1:2] * (p0_ref[...] + p1_ref[...]) + yself_ref[...], 0.0)
    y2 = jnp.dot(h, w_ref[...], preferred_element_type=jnp.float32)
    y2s_ref[...] = y2 * sc[:, 0:1]
    y2self_ref[...] = y2 * sc[:, 2:3]


_tc_c = pl.pallas_call(
    _tc_c_body,
    grid=(_NB,),
    in_specs=[
        pl.BlockSpec((_R, _H1), lambda i: (i, 0)),
        pl.BlockSpec((_R, _H1), lambda i: (i + _NB, 0)),
        pl.BlockSpec((_R, _H1), lambda i: (i, 0)),
        pl.BlockSpec((_R, 4), lambda i: (i, 0)),
        pl.BlockSpec((_H1, _H2), lambda i: (0, 0)),
    ],
    out_specs=[
        pl.BlockSpec((_R, _H2), lambda i: (i, 0)),
        pl.BlockSpec((_R, _H2), lambda i: (i, 0)),
    ],
    out_shape=[
        jax.ShapeDtypeStruct((_N, _H2), jnp.float32),
        jax.ShapeDtypeStruct((_N, _H2), jnp.float32),
    ],
)


def _tc_d_body(p0_ref, p1_ref, y2self_ref, sc_ref, wg_ref, bg_ref,
               z_ref, head_ref):
    sc = sc_ref[...]
    z = sc[:, 1:2] * (p0_ref[...] + p1_ref[...]) + y2self_ref[...]
    z_ref[...] = z
    head_ref[...] = (
        jnp.dot(z, wg_ref[...], preferred_element_type=jnp.float32)
        + bg_ref[...])


_tc_d = pl.pallas_call(
    _tc_d_body,
    grid=(_NB,),
    in_specs=[
        pl.BlockSpec((_R, _H2), lambda i: (i, 0)),
        pl.BlockSpec((_R, _H2), lambda i: (i + _NB, 0)),
        pl.BlockSpec((_R, _H2), lambda i: (i, 0)),
        pl.BlockSpec((_R, 4), lambda i: (i, 0)),
        pl.BlockSpec((_H2, _H2), lambda i: (0, 0)),
        pl.BlockSpec((1, _H2), lambda i: (0, 0)),
    ],
    out_specs=[
        pl.BlockSpec((_R, _H2), lambda i: (i, 0)),
        pl.BlockSpec((_R, _H2), lambda i: (i, 0)),
    ],
    out_shape=[
        jax.ShapeDtypeStruct((_N, _H2), jnp.float32),
        jax.ShapeDtypeStruct((_N, _H2), jnp.float32),
    ],
)


_BM = 400


def _tc_rec_body(a_ref, b_ref, o_ref):
    o_ref[...] = lax.dot_general(
        a_ref[...], b_ref[...], (((1,), (1,)), ((), ())),
        preferred_element_type=jnp.float32)


_tc_rec = pl.pallas_call(
    _tc_rec_body,
    grid=(_N // _BM,),
    in_specs=[
        pl.BlockSpec((_BM, _H2), lambda i: (i, 0)),
        pl.BlockSpec((_N, _H2), lambda i: (0, 0)),
    ],
    out_specs=pl.BlockSpec((_BM, _N), lambda i: (i, 0)),
    out_shape=jax.ShapeDtypeStruct((_N, _N), jnp.float32),
)


def _encode(x, ei, wa, wb, wg, bg2):
    src = ei[0]
    dst = ei[1]
    hist = _sc_hist(src, dst)
    pt = hist.T
    ys, yself, scales = _tc_a(pt, x, wa)
    src_r = src.reshape(_NCHUNK, _CB)
    dst_r = dst.reshape(_NCHUNK, _CB)
    p1 = _spmm_h1(ys, src_r, dst_r)
    y2s, y2self = _tc_c(p1, p1, yself, scales, wb)
    p2 = _spmm_h2(y2s, src_r, dst_r)
    z, head = _tc_d(p2, p2, y2self, scales, wg, bg2)
    return z, head


def kernel(x1, edge_index1, x2, edge_index2, W1_1, W1_2, W2_1, W2_2, Wg, bg):
    bg2 = bg.reshape(1, _H2)
    z1, head1 = _encode(x1, edge_index1, W1_1, W1_2, Wg, bg2)
    z2, _ = _encode(x2, edge_index2, W2_1, W2_2, Wg, bg2)
    rec1 = _tc_rec(z1, z1).reshape(-1)
    rec2 = _tc_rec(z2, z2).reshape(-1)
    return rec1, rec2, head1


# speedup vs baseline: 11.0546x; 1.0112x over previous
import functools

import jax
import jax.numpy as jnp
from jax import lax
from jax.experimental import pallas as pl
from jax.experimental.pallas import tpu as pltpu
from jax.experimental.pallas import tpu_sc as plsc

_N = 10000
_E = 320000
_D = 128
_H1 = 32
_H2 = 16

_NC = 2
_NS = 16
_NW = _NC * _NS
_L = 16
_EPW = _E // _NW
_RPS = _N // _NS
_CB = 128
_NCHUNK = _E // _CB
_CPW = _NCHUNK // _NW
_XTRA = _NCHUNK % _NW
_ZR = 640
_ZLAST = _N - (_NS - 1) * _ZR
_NBUF = 8

_MESH = plsc.VectorSubcoreMesh(core_axis_name="c", subcore_axis_name="s")


@functools.partial(
    pl.kernel,
    out_type=jax.ShapeDtypeStruct((2 * _NW, _N), jnp.float32),
    mesh=_MESH,
    compiler_params=pltpu.CompilerParams(needs_layout_passes=False),
    scratch_types=[
        pltpu.VMEM((_EPW,), jnp.int32),
        pltpu.VMEM((_EPW,), jnp.int32),
        pltpu.VMEM((_N,), jnp.float32),
        pltpu.VMEM((_N,), jnp.float32),
    ],
)
def _sc_hist(src_hbm, dst_hbm, out_hbm, idx_s, idx_d, hist_s, hist_d):
    wid = lax.axis_index("s") * _NC + lax.axis_index("c")
    base = wid * _EPW
    pltpu.sync_copy(src_hbm.at[pl.ds(base, _EPW)], idx_s)
    pltpu.sync_copy(dst_hbm.at[pl.ds(base, _EPW)], idx_d)

    zeros = jnp.zeros((_L,), jnp.float32)

    def zbody(i, c):
        hist_s[pl.ds(i * _L, _L)] = zeros
        hist_d[pl.ds(i * _L, _L)] = zeros
        return c

    lax.fori_loop(0, _N // _L, zbody, 0)

    ones = jnp.ones((_L,), jnp.float32)

    def ebody(i, c):
        plsc.addupdate_scatter(hist_s, [idx_s[pl.ds(i * _L, _L)]], ones)
        plsc.addupdate_scatter(hist_d, [idx_d[pl.ds(i * _L, _L)]], ones)
        return c

    lax.fori_loop(0, _EPW // _L, ebody, 0)

    pltpu.sync_copy(hist_s, out_hbm.at[wid])
    pltpu.sync_copy(hist_d, out_hbm.at[_NW + wid])


def _make_spmm(f):
    @functools.partial(
        pl.kernel,
        out_type=jax.ShapeDtypeStruct((_NC * _N, f), jnp.float32),
        mesh=_MESH,
        compiler_params=pltpu.CompilerParams(
            needs_layout_passes=False, use_tc_tiling_on_sc=False),
        scratch_types=[
            pltpu.VMEM((_CPW + 1, _CB), jnp.int32),
            pltpu.VMEM((_CPW + 1, _CB), jnp.int32),
            pltpu.VMEM((_NBUF, _CB, f), jnp.float32),
            pltpu.VMEM((_ZR, f), jnp.float32),
            pltpu.VMEM_SHARED((_N, f), jnp.float32),
            [pltpu.SemaphoreType.DMA] * _NBUF,
            [pltpu.SemaphoreType.DMA] * _NBUF,
        ],
    )
    def spmm(ys_hbm, src_hbm, dst_hbm, out_hbm, sidx, didx,
             rows, zbuf, acc, gsem, ssem):
        cid = lax.axis_index("c")
        sid = lax.axis_index("s")
        wid = sid * _NC + cid

        zeros = jnp.zeros((_L,), jnp.float32)

        def zbody(r, c):
            for j in range(f // _L):
                zbuf[r, pl.ds(j * _L, _L)] = zeros
            return c

        lax.fori_loop(0, _ZR, zbody, 0)

        @pl.when(sid < _NS - 1)
        def _():
            pltpu.sync_copy(zbuf, acc.at[pl.ds(sid * _ZR, _ZR)])

        @pl.when(sid == _NS - 1)
        def _():
            pltpu.sync_copy(zbuf.at[pl.ds(0, _ZLAST)],
                            acc.at[pl.ds((_NS - 1) * _ZR, _ZLAST)])

        cnt = jnp.where(wid < _XTRA, _CPW + 1, _CPW)
        start = wid * _CPW + jnp.minimum(wid, _XTRA)
        pltpu.sync_copy(src_hbm.at[pl.ds(start, _CPW)],
                        sidx.at[pl.ds(0, _CPW)])
        pltpu.sync_copy(dst_hbm.at[pl.ds(start, _CPW)],
                        didx.at[pl.ds(0, _CPW)])

        @pl.when(wid < _XTRA)
        def _():
            pltpu.sync_copy(src_hbm.at[pl.ds(start + _CPW, 1)],
                            sidx.at[pl.ds(_CPW, 1)])
            pltpu.sync_copy(dst_hbm.at[pl.ds(start + _CPW, 1)],
                            didx.at[pl.ds(_CPW, 1)])

        plsc.subcore_barrier()

        def gather(k, b):
            return pltpu.make_async_copy(
                ys_hbm.at[sidx.at[k]], rows.at[b], gsem[b])

        def scatter(k, b):
            return pltpu.make_async_copy(
                rows.at[b], acc.at[didx.at[k]], ssem[b])

        def issue(k, b):
            @pl.when(k < cnt)
            def _():
                @pl.when(k >= _NBUF)
                def _():
                    scatter(k - _NBUF, b).wait()
                gather(k, b).start()

        def drain(k, b):
            @pl.when(k < cnt)
            def _():
                gather(k, b).wait()
                pltpu.async_copy(rows.at[b], acc.at[didx.at[k]],
                                 ssem[b], add=True)

        for b in range(_NBUF):
            issue(b, b)

        def ebody(i, c):
            k0 = _NBUF * i
            for b in range(_NBUF):
                drain(k0 + b, b)
                issue(k0 + b + _NBUF, b)
            return c

        lax.fori_loop(0, (_CPW + _NBUF) // _NBUF, ebody, 0)

        for b in range(_NBUF):
            scatter(0, b).wait()

        plsc.subcore_barrier()

        @pl.when(sid < _NS - 1)
        def _():
            pltpu.sync_copy(acc.at[pl.ds(sid * _ZR, _ZR)],
                            out_hbm.at[pl.ds(cid * _N + sid * _ZR, _ZR)])

        @pl.when(sid == _NS - 1)
        def _():
            pltpu.sync_copy(
                acc.at[pl.ds((_NS - 1) * _ZR, _ZLAST)],
                out_hbm.at[pl.ds(cid * _N + (_NS - 1) * _ZR, _ZLAST)])

    return spmm


_spmm_h1 = _make_spmm(_H1)
_spmm_h2 = _make_spmm(_H2)


_R = 2000
_NB = _N // _R


def _tc_a_body(pt_ref, x_ref, w_ref, ys_ref, yself_ref, sc_ref):
    pt = pt_ref[...]
    deg_o = jnp.sum(pt[:, :_NW], axis=1, keepdims=True) + 1.0
    deg_i = jnp.sum(pt[:, _NW:], axis=1, keepdims=True) + 1.0
    r_o = lax.rsqrt(deg_o)
    r_i = lax.rsqrt(deg_i)
    s = r_o * r_i
    y = jnp.dot(x_ref[...], w_ref[...], preferred_element_type=jnp.float32)
    ys_ref[...] = y * r_o
    yself_ref[...] = y * s
    sc_ref[...] = jnp.concatenate([r_o, r_i, s, jnp.zeros_like(s)], axis=1)


_tc_a = pl.pallas_call(
    _tc_a_body,
    grid=(_NB,),
    in_specs=[
        pl.BlockSpec((_R, 2 * _NW), lambda i: (i, 0)),
        pl.BlockSpec((_R, _D), lambda i: (i, 0)),
        pl.BlockSpec((_D, _H1), lambda i: (0, 0)),
    ],
    out_specs=[
        pl.BlockSpec((_R, _H1), lambda i: (i, 0)),
        pl.BlockSpec((_R, _H1), lambda i: (i, 0)),
        pl.BlockSpec((_R, 4), lambda i: (i, 0)),
    ],
    out_shape=[
        jax.ShapeDtypeStruct((_N, _H1), jnp.float32),
        jax.ShapeDtypeStruct((_N, _H1), jnp.float32),
        jax.ShapeDtypeStruct((_N, 4), jnp.float32),
    ],
)


def _tc_c_body(p0_ref, p1_ref, yself_ref, sc_ref, w_ref, y2s_ref, y2self_ref):
    sc = sc_ref[...]
    h = jnp.maximum(
        sc[:, 1:2] * (p0_ref[...] + p1_ref[...]) + yself_ref[...], 0.0)
    y2 = jnp.dot(h, w_ref[...], preferred_element_type=jnp.float32)
    y2s_ref[...] = y2 * sc[:, 0:1]
    y2self_ref[...] = y2 * sc[:, 2:3]


_tc_c = pl.pallas_call(
    _tc_c_body,
    grid=(_NB,),
    in_specs=[
        pl.BlockSpec((_R, _H1), lambda i: (i, 0)),
        pl.BlockSpec((_R, _H1), lambda i: (i + _NB, 0)),
        pl.BlockSpec((_R, _H1), lambda i: (i, 0)),
        pl.BlockSpec((_R, 4), lambda i: (i, 0)),
        pl.BlockSpec((_H1, _H2), lambda i: (0, 0)),
    ],
    out_specs=[
        pl.BlockSpec((_R, _H2), lambda i: (i, 0)),
        pl.BlockSpec((_R, _H2), lambda i: (i, 0)),
    ],
    out_shape=[
        jax.ShapeDtypeStruct((_N, _H2), jnp.float32),
        jax.ShapeDtypeStruct((_N, _H2), jnp.float32),
    ],
)


def _tc_d_body(p0_ref, p1_ref, y2self_ref, sc_ref, wg_ref, bg_ref,
               z_ref, head_ref):
    sc = sc_ref[...]
    z = sc[:, 1:2] * (p0_ref[...] + p1_ref[...]) + y2self_ref[...]
    z_ref[...] = z
    head_ref[...] = (
        jnp.dot(z, wg_ref[...], preferred_element_type=jnp.float32)
        + bg_ref[...])


_tc_d = pl.pallas_call(
    _tc_d_body,
    grid=(_NB,),
    in_specs=[
        pl.BlockSpec((_R, _H2), lambda i: (i, 0)),
        pl.BlockSpec((_R, _H2), lambda i: (i + _NB, 0)),
        pl.BlockSpec((_R, _H2), lambda i: (i, 0)),
        pl.BlockSpec((_R, 4), lambda i: (i, 0)),
        pl.BlockSpec((_H2, _H2), lambda i: (0, 0)),
        pl.BlockSpec((1, _H2), lambda i: (0, 0)),
    ],
    out_specs=[
        pl.BlockSpec((_R, _H2), lambda i: (i, 0)),
        pl.BlockSpec((_R, _H2), lambda i: (i, 0)),
    ],
    out_shape=[
        jax.ShapeDtypeStruct((_N, _H2), jnp.float32),
        jax.ShapeDtypeStruct((_N, _H2), jnp.float32),
    ],
)


_BM = 400


def _tc_rec_body(a_ref, b_ref, o_ref):
    o_ref[...] = lax.dot_general(
        a_ref[...], b_ref[...], (((1,), (1,)), ((), ())),
        preferred_element_type=jnp.float32)


_tc_rec = pl.pallas_call(
    _tc_rec_body,
    grid=(_N // _BM,),
    in_specs=[
        pl.BlockSpec((_BM, _H2), lambda i: (i, 0)),
        pl.BlockSpec((_N, _H2), lambda i: (0, 0)),
    ],
    out_specs=pl.BlockSpec((_BM, _N), lambda i: (i, 0)),
    out_shape=jax.ShapeDtypeStruct((_N, _N), jnp.float32),
)


def _encode(x, ei, wa, wb, wg, bg2):
    src = ei[0]
    dst = ei[1]
    hist = _sc_hist(src, dst)
    pt = hist.T
    ys, yself, scales = _tc_a(pt, x, wa)
    src_r = src.reshape(_NCHUNK, _CB)
    dst_r = dst.reshape(_NCHUNK, _CB)
    p1 = _spmm_h1(ys, src_r, dst_r)
    y2s, y2self = _tc_c(p1, p1, yself, scales, wb)
    p2 = _spmm_h2(y2s, src_r, dst_r)
    z, head = _tc_d(p2, p2, y2self, scales, wg, bg2)
    return z, head


def kernel(x1, edge_index1, x2, edge_index2, W1_1, W1_2, W2_1, W2_2, Wg, bg):
    bg2 = bg.reshape(1, _H2)
    z1, head1 = _encode(x1, edge_index1, W1_1, W1_2, Wg, bg2)
    z2, _ = _encode(x2, edge_index2, W2_1, W2_2, Wg, bg2)
    rec1 = _tc_rec(z1, z1).reshape(-1)
    rec2 = _tc_rec(z2, z2).reshape(-1)
    return rec1, rec2, head1
